# baseline (speedup 1.0000x reference)
import functools

import jax
import jax.numpy as jnp
from jax import lax
from jax.experimental import pallas as pl
from jax.experimental.pallas import tpu as pltpu
from jax.experimental.pallas import tpu_sc as plsc

N_OUT = 100000
CT = 256
NSWEEP = 13
NCHT = 416
DUMMY = CT
ACC_ROWS = CT + 8
K = 96
SB = 480
EPAD = SB + 8
NLANE = 16
BND_PAD = 448


def _sc_body(ll0T, ll1T, ip0, vp0, ip1, vp1, bnd, zsrc, out,
             val_sb, idx_sb, rows_a, rows_b, bnd_v, acc,
             sem_a, sem_b, zsem):
    B = rows_a.shape[1]
    c = lax.axis_index("c")
    s = lax.axis_index("s")
    wid = c * 16 + s

    pltpu.sync_copy(bnd, bnd_v)

    lanes = lax.iota(jnp.int32, NLANE)
    cols = [lanes + NLANE * g for g in range(B // NLANE)]

    def extract(j):
        w = (j // NLANE) * NLANE
        seg = bnd_v[pl.ds(w, NLANE)]
        return jnp.sum(jnp.where(lanes == (j - w), seg, 0))

    def accumulate(rows_buf, boff, sa, lo, hi, base):
        for j in range(K // NLANE):
            iv = idx_sb[pl.ds(boff + NLANE * j, NLANE)]
            pos = (sa + NLANE * j) + boff + lanes
            msk = (pos >= lo) & (pos < hi)
            lidx = jnp.where(msk, iv - base, DUMMY)

            def edge(l, carry, j=j, lidx=lidx):
                rsp = jnp.take_along_axis(
                    lidx, jnp.full((NLANE,), l, jnp.int32), axis=0)
                e = NLANE * j + l
                for g in range(B // NLANE):
                    v = rows_buf[e, pl.ds(NLANE * g, NLANE)]
                    plsc.addupdate_scatter(acc, [rsp, cols[g]], v)
                return carry

            lax.fori_loop(0, NLANE, edge, 0)

    def do_chunk(sweep, carry):
        m = wid * NSWEEP + sweep
        base = m * CT

        zcp = pltpu.async_copy(zsrc, acc.at[pl.ds(0, CT)], zsem)

        for child in range(2):
            llT = ll0T if child == 0 else ll1T
            ip = ip0 if child == 0 else ip1
            vp = vp0 if child == 0 else vp1
            lo = extract(child * BND_PAD + m)
            hi = extract(child * BND_PAD + m + 1)
            a0 = (lo // 8) * 8
            nsb = (hi - a0 + SB - 1) // SB
            if child == 0:
                zcp.wait()

            def superbatch(sb, carry2, ip=ip, vp=vp, llT=llT,
                           lo=lo, hi=hi, a0=a0):
                sa = a0 + SB * sb
                pltpu.sync_copy(vp.at[pl.ds(sa, SB)], val_sb)
                pltpu.sync_copy(ip.at[pl.ds(sa, SB)], idx_sb)
                rem = hi - sa
                nbi = jnp.minimum(SB // K, (rem + K - 1) // K)
                pltpu.async_copy(
                    llT.at[val_sb.at[pl.ds(0, K)]], rows_a, sem_a)

                def pair(t2, carry3, llT=llT, sa=sa, lo=lo, hi=hi,
                         nbi=nbi):
                    b0 = 2 * t2
                    b1 = 2 * t2 + 1
                    pltpu.make_async_copy(
                        llT.at[val_sb.at[pl.ds(0, K)]], rows_a,
                        sem_a).wait()

                    @pl.when(b1 < nbi)
                    def _():
                        pltpu.async_copy(
                            llT.at[val_sb.at[pl.ds(K * b1, K)]],
                            rows_b, sem_b)
                    accumulate(rows_a, K * b0, sa, lo, hi, base)

                    @pl.when(b1 < nbi)
                    def _():
                        pltpu.make_async_copy(
                            llT.at[val_sb.at[pl.ds(0, K)]], rows_b,
                            sem_b).wait()

                        @pl.when(b1 + 1 < nbi)
                        def _():
                            pltpu.async_copy(
                                llT.at[val_sb.at[pl.ds(K * (b1 + 1), K)]],
                                rows_a, sem_a)
                        accumulate(rows_b, K * b1, sa, lo, hi, base)
                    return carry3

                lax.fori_loop(0, (nbi + 1) // 2, pair, 0)
                return carry2

            lax.fori_loop(0, nsb, superbatch, 0)

        pltpu.sync_copy(acc.at[pl.ds(0, CT)], out.at[pl.ds(base, CT)])
        return carry

    lax.fori_loop(0, NSWEEP, do_chunk, 0)


def kernel(ll0, ll1, edge_indices0, edge_values0, edge_indices1, edge_values1):
    Bb, M = ll0.shape
    E = edge_indices0.shape[0]

    ll0T = ll0.T
    ll1T = ll1.T

    starts = jnp.arange(NCHT, dtype=jnp.int32) * CT

    def prep(idx, val):
        b = jnp.searchsorted(idx, starts).astype(jnp.int32)
        b = jnp.concatenate(
            [b, jnp.full((BND_PAD - NCHT,), E, dtype=jnp.int32)])
        pad = jnp.zeros((EPAD,), dtype=jnp.int32)
        return b, jnp.concatenate([idx, pad]), jnp.concatenate([val, pad])

    b0, ip0, vp0 = prep(edge_indices0, edge_values0)
    b1, ip1, vp1 = prep(edge_indices1, edge_values1)
    bnd_all = jnp.concatenate([b0, b1])
    zsrc = jnp.zeros((CT, Bb), dtype=jnp.float32)

    sc_call = functools.partial(
        pl.kernel,
        mesh=plsc.VectorSubcoreMesh(core_axis_name="c", subcore_axis_name="s"),
        compiler_params=pltpu.CompilerParams(needs_layout_passes=False),
        out_type=jax.ShapeDtypeStruct((NCHT * CT, Bb), jnp.float32),
        scratch_types=[
            pltpu.VMEM((SB,), jnp.int32),
            pltpu.VMEM((SB,), jnp.int32),
            pltpu.VMEM((K, Bb), jnp.float32),
            pltpu.VMEM((K, Bb), jnp.float32),
            pltpu.VMEM((2 * BND_PAD,), jnp.int32),
            pltpu.VMEM((ACC_ROWS, Bb), jnp.float32),
            pltpu.SemaphoreType.DMA,
            pltpu.SemaphoreType.DMA,
            pltpu.SemaphoreType.DMA,
        ],
    )(_sc_body)

    outT = sc_call(ll0T, ll1T, ip0, vp0, ip1, vp1, bnd_all, zsrc)
    return outT[:N_OUT].T


# speedup vs baseline: 1.0663x; 1.0663x over previous
import functools

import jax
import jax.numpy as jnp
from jax import lax
from jax.experimental import pallas as pl
from jax.experimental.pallas import tpu as pltpu
from jax.experimental.pallas import tpu_sc as plsc

N_OUT = 100000
CT = 256
NSWEEP = 13
NCHT = 416
DUMMY = CT
ACC_ROWS = CT + 8
K = 64
SB = 512
EPAD = SB + 8
NLANE = 16
BND_PAD = 448


def _sc_body(ll0T, ll1T, ip0, vp0, ip1, vp1, bnd, zsrc, out,
             val_sb, idx_sb, rows_a, rows_b, bnd_v, acc,
             sem_a, sem_b, zsem):
    B = rows_a.shape[1]
    c = lax.axis_index("c")
    s = lax.axis_index("s")
    wid = c * 16 + s

    pltpu.sync_copy(bnd, bnd_v)

    lanes = lax.iota(jnp.int32, NLANE)
    cols = [lanes + NLANE * g for g in range(B // NLANE)]

    def extract(j):
        w = (j // NLANE) * NLANE
        seg = bnd_v[pl.ds(w, NLANE)]
        return jnp.sum(jnp.where(lanes == (j - w), seg, 0))

    def accumulate(rows_buf, boff, sa, lo, hi, base):
        for j in range(K // NLANE):
            iv = idx_sb[pl.ds(boff + NLANE * j, NLANE)]
            pos = (sa + NLANE * j) + boff + lanes
            msk = (pos >= lo) & (pos < hi)
            lidx = jnp.where(msk, iv - base, DUMMY)

            def edge(l, carry, j=j, lidx=lidx):
                rsp = jnp.take_along_axis(
                    lidx, jnp.full((NLANE,), l, jnp.int32), axis=0)
                e = NLANE * j + l
                for g in range(B // NLANE):
                    v = rows_buf[e, pl.ds(NLANE * g, NLANE)]
                    plsc.addupdate_scatter(acc, [rsp, cols[g]], v)
                return carry

            lax.fori_loop(0, NLANE, edge, 0)

    def do_chunk(sweep, carry):
        m = wid * NSWEEP + sweep
        base = m * CT

        zcp = pltpu.async_copy(zsrc, acc.at[pl.ds(0, CT)], zsem)

        for child in range(2):
            llT = ll0T if child == 0 else ll1T
            ip = ip0 if child == 0 else ip1
            vp = vp0 if child == 0 else vp1
            lo = extract(child * BND_PAD + m)
            hi = extract(child * BND_PAD + m + 1)
            a0 = (lo // 8) * 8
            nsb = (hi - a0 + SB - 1) // SB

            def superbatch(sb, carry2, ip=ip, vp=vp, llT=llT,
                           lo=lo, hi=hi, a0=a0, child=child):
                sa = a0 + SB * sb
                pltpu.sync_copy(vp.at[pl.ds(sa, SB)], val_sb)
                pltpu.sync_copy(ip.at[pl.ds(sa, SB)], idx_sb)
                rem = hi - sa
                nbi = jnp.minimum(SB // K, (rem + K - 1) // K)
                pltpu.async_copy(
                    llT.at[val_sb.at[pl.ds(0, K)]], rows_a, sem_a)
                if child == 0:
                    @pl.when(sb == 0)
                    def _():
                        zcp.wait()

                def pair(t2, carry3, llT=llT, sa=sa, lo=lo, hi=hi,
                         nbi=nbi):
                    b0 = 2 * t2
                    b1 = 2 * t2 + 1
                    pltpu.make_async_copy(
                        llT.at[val_sb.at[pl.ds(0, K)]], rows_a,
                        sem_a).wait()

                    @pl.when(b1 < nbi)
                    def _():
                        pltpu.async_copy(
                            llT.at[val_sb.at[pl.ds(K * b1, K)]],
                            rows_b, sem_b)
                    accumulate(rows_a, K * b0, sa, lo, hi, base)

                    @pl.when(b1 < nbi)
                    def _():
                        pltpu.make_async_copy(
                            llT.at[val_sb.at[pl.ds(0, K)]], rows_b,
                            sem_b).wait()

                        @pl.when(b1 + 1 < nbi)
                        def _():
                            pltpu.async_copy(
                                llT.at[val_sb.at[pl.ds(K * (b1 + 1), K)]],
                                rows_a, sem_a)
                        accumulate(rows_b, K * b1, sa, lo, hi, base)
                    return carry3

                lax.fori_loop(0, (nbi + 1) // 2, pair, 0)
                return carry2

            lax.fori_loop(0, nsb, superbatch, 0)
            if child == 0:
                @pl.when(nsb == 0)
                def _():
                    zcp.wait()

        pltpu.sync_copy(acc.at[pl.ds(0, CT)], out.at[pl.ds(base, CT)])
        return carry

    lax.fori_loop(0, NSWEEP, do_chunk, 0)


def kernel(ll0, ll1, edge_indices0, edge_values0, edge_indices1, edge_values1):
    Bb, M = ll0.shape
    E = edge_indices0.shape[0]

    ll0T = ll0.T
    ll1T = ll1.T

    starts = jnp.arange(NCHT, dtype=jnp.int32) * CT

    def prep(idx, val):
        b = jnp.searchsorted(idx, starts).astype(jnp.int32)
        b = jnp.concatenate(
            [b, jnp.full((BND_PAD - NCHT,), E, dtype=jnp.int32)])
        pad = jnp.zeros((EPAD,), dtype=jnp.int32)
        return b, jnp.concatenate([idx, pad]), jnp.concatenate([val, pad])

    b0, ip0, vp0 = prep(edge_indices0, edge_values0)
    b1, ip1, vp1 = prep(edge_indices1, edge_values1)
    bnd_all = jnp.concatenate([b0, b1])
    zsrc = jnp.zeros((CT, Bb), dtype=jnp.float32)

    sc_call = functools.partial(
        pl.kernel,
        mesh=plsc.VectorSubcoreMesh(core_axis_name="c", subcore_axis_name="s"),
        compiler_params=pltpu.CompilerParams(needs_layout_passes=False),
        out_type=jax.ShapeDtypeStruct((NCHT * CT, Bb), jnp.float32),
        scratch_types=[
            pltpu.VMEM((SB,), jnp.int32),
            pltpu.VMEM((SB,), jnp.int32),
            pltpu.VMEM((K, Bb), jnp.float32),
            pltpu.VMEM((K, Bb), jnp.float32),
            pltpu.VMEM((2 * BND_PAD,), jnp.int32),
            pltpu.VMEM((ACC_ROWS, Bb), jnp.float32),
            pltpu.SemaphoreType.DMA,
            pltpu.SemaphoreType.DMA,
            pltpu.SemaphoreType.DMA,
        ],
    )(_sc_body)

    outT = sc_call(ll0T, ll1T, ip0, vp0, ip1, vp1, bnd_all, zsrc)
    return outT[:N_OUT].T


# speedup vs baseline: 1.5687x; 1.4711x over previous
import functools

import jax
import jax.numpy as jnp
from jax import lax
from jax.experimental import pallas as pl
from jax.experimental.pallas import tpu as pltpu
from jax.experimental.pallas import tpu_sc as plsc

N_OUT = 100000
CT = 256
NSWEEP = 13
NCHT = 416
DUMMY = CT
ACC_ROWS = CT + 8
K = 64
SB = 512
EPAD = SB + 8
NLANE = 16
BND_PAD = 448


def _sc_body(ll0T, ll1T, ip0, vp0, ip1, vp1, bnd, zsrc, out,
             val_sb, idx_sb, rows_a, rows_b, bnd_v, acc,
             sem_a, sem_b, zsem):
    B = rows_a.shape[1]
    c = lax.axis_index("c")
    s = lax.axis_index("s")
    wid = c * 16 + s

    pltpu.sync_copy(bnd, bnd_v)

    lanes = lax.iota(jnp.int32, NLANE)
    cols = [lanes + NLANE * g for g in range(B // NLANE)]

    def extract(j):
        w = (j // NLANE) * NLANE
        seg = bnd_v[pl.ds(w, NLANE)]
        return jnp.sum(jnp.where(lanes == (j - w), seg, 0))

    def accumulate(rows_buf, boff, sa, lo, hi, base):
        for j in range(K // NLANE):
            iv = idx_sb[pl.ds(boff + NLANE * j, NLANE)]
            pos = (sa + NLANE * j) + boff + lanes
            msk = (pos >= lo) & (pos < hi)
            lidx = jnp.where(msk, iv - base, DUMMY)

            def edge(l, carry, j=j, lidx=lidx):
                rsp = jnp.take_along_axis(
                    lidx, jnp.full((NLANE,), l, jnp.int32), axis=0)
                e = NLANE * j + l
                vs = [rows_buf[e, pl.ds(NLANE * g, NLANE)]
                      for g in range(B // NLANE)]
                for g in range(B // NLANE):
                    plsc.addupdate_scatter(acc, [rsp, cols[g]], vs[g])
                return carry

            lax.fori_loop(0, NLANE, edge, 0)

    def do_chunk(sweep, carry):
        m = wid * NSWEEP + sweep
        base = m * CT

        zcp = pltpu.async_copy(zsrc, acc.at[pl.ds(0, CT)], zsem)

        for child in range(2):
            llT = ll0T if child == 0 else ll1T
            ip = ip0 if child == 0 else ip1
            vp = vp0 if child == 0 else vp1
            lo = extract(child * BND_PAD + m)
            hi = extract(child * BND_PAD + m + 1)
            a0 = (lo // 8) * 8
            nsb = (hi - a0 + SB - 1) // SB

            def superbatch(sb, carry2, ip=ip, vp=vp, llT=llT,
                           lo=lo, hi=hi, a0=a0, child=child):
                sa = a0 + SB * sb
                pltpu.sync_copy(vp.at[pl.ds(sa, SB)], val_sb)
                pltpu.sync_copy(ip.at[pl.ds(sa, SB)], idx_sb)
                rem = hi - sa
                nbi = jnp.minimum(SB // K, (rem + K - 1) // K)
                pltpu.async_copy(
                    llT.at[val_sb.at[pl.ds(0, K)]], rows_a, sem_a)
                if child == 0:
                    @pl.when(sb == 0)
                    def _():
                        zcp.wait()

                def pair(t2, carry3, llT=llT, sa=sa, lo=lo, hi=hi,
                         nbi=nbi):
                    b0 = 2 * t2
                    b1 = 2 * t2 + 1
                    pltpu.make_async_copy(
                        llT.at[val_sb.at[pl.ds(0, K)]], rows_a,
                        sem_a).wait()

                    @pl.when(b1 < nbi)
                    def _():
                        pltpu.async_copy(
                            llT.at[val_sb.at[pl.ds(K * b1, K)]],
                            rows_b, sem_b)
                    accumulate(rows_a, K * b0, sa, lo, hi, base)

                    @pl.when(b1 < nbi)
                    def _():
                        pltpu.make_async_copy(
                            llT.at[val_sb.at[pl.ds(0, K)]], rows_b,
                            sem_b).wait()

                        @pl.when(b1 + 1 < nbi)
                        def _():
                            pltpu.async_copy(
                                llT.at[val_sb.at[pl.ds(K * (b1 + 1), K)]],
                                rows_a, sem_a)
                        accumulate(rows_b, K * b1, sa, lo, hi, base)
                    return carry3

                lax.fori_loop(0, (nbi + 1) // 2, pair, 0)
                return carry2

            lax.fori_loop(0, nsb, superbatch, 0)
            if child == 0:
                @pl.when(nsb == 0)
                def _():
                    zcp.wait()

        pltpu.sync_copy(acc.at[pl.ds(0, CT)], out.at[pl.ds(base, CT)])
        return carry

    lax.fori_loop(0, NSWEEP, do_chunk, 0)


def kernel(ll0, ll1, edge_indices0, edge_values0, edge_indices1, edge_values1):
    Bb, M = ll0.shape
    E = edge_indices0.shape[0]

    ll0T = ll0.T
    ll1T = ll1.T

    starts = jnp.arange(NCHT, dtype=jnp.int32) * CT

    def prep(idx, val):
        b = jnp.searchsorted(idx, starts).astype(jnp.int32)
        b = jnp.concatenate(
            [b, jnp.full((BND_PAD - NCHT,), E, dtype=jnp.int32)])
        pad = jnp.zeros((EPAD,), dtype=jnp.int32)
        return b, jnp.concatenate([idx, pad]), jnp.concatenate([val, pad])

    b0, ip0, vp0 = prep(edge_indices0, edge_values0)
    b1, ip1, vp1 = prep(edge_indices1, edge_values1)
    bnd_all = jnp.concatenate([b0, b1])
    zsrc = jnp.zeros((CT, Bb), dtype=jnp.float32)

    sc_call = functools.partial(
        pl.kernel,
        mesh=plsc.VectorSubcoreMesh(core_axis_name="c", subcore_axis_name="s"),
        compiler_params=pltpu.CompilerParams(needs_layout_passes=False),
        out_type=jax.ShapeDtypeStruct((NCHT * CT, Bb), jnp.float32),
        scratch_types=[
            pltpu.VMEM((SB,), jnp.int32),
            pltpu.VMEM((SB,), jnp.int32),
            pltpu.VMEM((K, Bb), jnp.float32),
            pltpu.VMEM((K, Bb), jnp.float32),
            pltpu.VMEM((2 * BND_PAD,), jnp.int32),
            pltpu.VMEM((ACC_ROWS, Bb), jnp.float32),
            pltpu.SemaphoreType.DMA,
            pltpu.SemaphoreType.DMA,
            pltpu.SemaphoreType.DMA,
        ],
    )(_sc_body)

    outT = sc_call(ll0T, ll1T, ip0, vp0, ip1, vp1, bnd_all, zsrc)
    return outT[:N_OUT].T
